# Initial kernel scaffold; baseline (speedup 1.0000x reference)
import jax
import jax.numpy as jnp
from jax.experimental import pallas as pl


def kernel(c, e, Wc, We, ba, Wa):
    raise NotImplementedError("write your pallas kernel here")


# speedup vs baseline: 3.1897x; 3.1897x over previous
import functools

import jax
import jax.numpy as jnp
from jax.experimental import pallas as pl

B = 16384
D_CONTEXT = 2048
D_TASK = 1024
D_ATTN = 1024
N_MODULES = 64
K_ACTIVE = 8

TILE = 512

_DOT_KW = dict(preferred_element_type=jnp.float32)


def _gater_body(c_ref, e_ref, Wc_ref, We_ref, ba_ref, Wa_ref,
                phi_ref, phik_ref, idx_ref):
    pre = (jax.lax.dot_general(c_ref[...], Wc_ref[...],
                               (((1,), (1,)), ((), ())), **_DOT_KW)
           + jax.lax.dot_general(e_ref[...], We_ref[...],
                                 (((1,), (1,)), ((), ())), **_DOT_KW)
           + ba_ref[...])
    h = jnp.tanh(pre)
    A = jax.lax.dot_general(h, Wa_ref[...], (((1,), (1,)), ((), ())),
                            **_DOT_KW)
    m = jnp.max(A, axis=1, keepdims=True)
    ex = jnp.exp(A - m)
    phi = ex / jnp.sum(ex, axis=1, keepdims=True)
    phi_ref[...] = phi

    iota64 = jax.lax.broadcasted_iota(jnp.int32, (TILE, N_MODULES), 1)
    iota8 = jax.lax.broadcasted_iota(jnp.int32, (TILE, K_ACTIVE), 1)
    run = phi
    total_mask = jnp.zeros((TILE, N_MODULES), jnp.bool_)
    ssum = jnp.zeros((TILE, 1), jnp.float32)
    idx_out = jnp.zeros((TILE, K_ACTIVE), jnp.int32)
    for k in range(K_ACTIVE):
        mx = jnp.max(run, axis=1, keepdims=True)
        sel = jnp.min(jnp.where(run == mx, iota64, N_MODULES),
                      axis=1, keepdims=True)
        onehot = iota64 == sel
        total_mask = jnp.logical_or(total_mask, onehot)
        ssum = ssum + mx
        idx_out = jnp.where(iota8 == k, sel, idx_out)
        run = jnp.where(onehot, -jnp.inf, run)

    phik_ref[...] = jnp.where(total_mask, phi, 0.0) / (ssum + 1e-8)
    idx_ref[...] = idx_out


@jax.jit
def kernel(c, e, Wc, We, ba, Wa):
    grid = (B // TILE,)
    phi, phi_k, idx = pl.pallas_call(
        _gater_body,
        grid=grid,
        in_specs=[
            pl.BlockSpec((TILE, D_CONTEXT), lambda i: (i, 0)),
            pl.BlockSpec((TILE, D_TASK), lambda i: (i, 0)),
            pl.BlockSpec((D_ATTN, D_CONTEXT), lambda i: (0, 0)),
            pl.BlockSpec((D_ATTN, D_TASK), lambda i: (0, 0)),
            pl.BlockSpec((1, D_ATTN), lambda i: (0, 0)),
            pl.BlockSpec((N_MODULES, D_ATTN), lambda i: (0, 0)),
        ],
        out_specs=[
            pl.BlockSpec((TILE, N_MODULES), lambda i: (i, 0)),
            pl.BlockSpec((TILE, N_MODULES), lambda i: (i, 0)),
            pl.BlockSpec((TILE, K_ACTIVE), lambda i: (i, 0)),
        ],
        out_shape=[
            jax.ShapeDtypeStruct((B, N_MODULES), jnp.float32),
            jax.ShapeDtypeStruct((B, N_MODULES), jnp.float32),
            jax.ShapeDtypeStruct((B, K_ACTIVE), jnp.int32),
        ],
    )(c, e, Wc, We, ba.reshape(1, D_ATTN), Wa)
    return (phi, phi_k, idx)
